# baseline (speedup 1.0000x reference)
import functools

import jax
import jax.numpy as jnp
from jax import lax
from jax.experimental import pallas as pl
from jax.experimental.pallas import tpu as pltpu
from jax.experimental.pallas import tpu_sc as plsc

VOCAB = 1_000_000
EMBED_DIM = 64
NUM_CLASSES = 100
BATCH = 4096
SEQ_LEN = 200
SLICE_A = 104
SLICE_B = 96

NUM_CORES = 2
NUM_SUBCORES = 16
NUM_WORKERS = NUM_CORES * NUM_SUBCORES
B_PER_W = BATCH // NUM_WORKERS
LANES = 16
W32 = EMBED_DIM // 2

_mesh = plsc.VectorSubcoreMesh(core_axis_name="c", subcore_axis_name="s")


@functools.partial(
    pl.kernel,
    mesh=_mesh,
    compiler_params=pltpu.CompilerParams(use_tc_tiling_on_sc=False),
    out_type=jax.ShapeDtypeStruct((BATCH, EMBED_DIM), jnp.float32),
    scratch_types=[
        pltpu.VMEM((B_PER_W, SEQ_LEN), jnp.int32),
        pltpu.VMEM((2, SEQ_LEN, EMBED_DIM), jnp.bfloat16),
        pltpu.VMEM((B_PER_W, EMBED_DIM), jnp.float32),
        pltpu.SemaphoreType.DMA,
        pltpu.SemaphoreType.DMA,
    ],
)
def _pool(tok_hbm, table_hbm, out_hbm, idx_v, rows_v, out_v, sem0, sem1):
    wid = lax.axis_index("s") * NUM_CORES + lax.axis_index("c")
    base = wid * B_PER_W
    sems = (sem0, sem1)

    pltpu.sync_copy(tok_hbm.at[pl.ds(base, B_PER_W)], idx_v)

    def issue(u, buf):
        pltpu.async_copy(
            table_hbm.at[idx_v.at[u, pl.ds(0, SLICE_A)]],
            rows_v.at[buf, pl.ds(0, SLICE_A)],
            sems[buf],
        )
        pltpu.async_copy(
            table_hbm.at[idx_v.at[u, pl.ds(SLICE_A, SLICE_B)]],
            rows_v.at[buf, pl.ds(SLICE_A, SLICE_B)],
            sems[buf],
        )

    def drain(u, buf):
        pltpu.make_async_copy(
            table_hbm.at[idx_v.at[u, pl.ds(0, SLICE_A)]],
            rows_v.at[buf, pl.ds(0, SLICE_A)],
            sems[buf],
        ).wait()
        pltpu.make_async_copy(
            table_hbm.at[idx_v.at[u, pl.ds(SLICE_A, SLICE_B)]],
            rows_v.at[buf, pl.ds(SLICE_A, SLICE_B)],
            sems[buf],
        ).wait()

    inv = jnp.float32(1.0 / SEQ_LEN)
    zero = jnp.zeros((2, LANES), jnp.float32)

    issue(0, 0)
    issue(1, 1)

    def outer(u0):
        for buf in range(2):
            u = u0 + buf
            drain(u, buf)
            rows_b = rows_v.at[buf]

            def acc_body(r, carry):
                a0, a1 = carry
                lo = rows_b[r, pl.ds(0, 2 * LANES)].reshape(2, LANES)
                hi = rows_b[r, pl.ds(2 * LANES, 2 * LANES)].reshape(2, LANES)
                return (a0 + lo.astype(jnp.float32), a1 + hi.astype(jnp.float32))

            acc = lax.fori_loop(0, SEQ_LEN, acc_body, (zero, zero))
            for k in range(4):
                out_v[u, pl.ds(k * LANES, LANES)] = acc[k // 2][k % 2] * inv

            @pl.when(u + 2 < B_PER_W)
            def _():
                issue(u + 2, buf)

    lax.fori_loop(0, B_PER_W // 2, lambda i, _: (outer(2 * i), 0)[1], 0)

    pltpu.sync_copy(out_v, out_hbm.at[pl.ds(base, B_PER_W)])


def _head_body(p_ref, w_ref, b_ref, o_ref):
    o_ref[...] = (
        jnp.dot(p_ref[...], w_ref[...], preferred_element_type=jnp.float32)
        + b_ref[...]
    )


def _head(pooled, Wp, b):
    blk = 512
    return pl.pallas_call(
        _head_body,
        out_shape=jax.ShapeDtypeStruct((BATCH, NUM_CLASSES), jnp.float32),
        grid=(BATCH // blk,),
        in_specs=[
            pl.BlockSpec((blk, EMBED_DIM), lambda i: (i, 0)),
            pl.BlockSpec((EMBED_DIM, NUM_CLASSES), lambda i: (0, 0)),
            pl.BlockSpec((1, NUM_CLASSES), lambda i: (0, 0)),
        ],
        out_specs=pl.BlockSpec((blk, NUM_CLASSES), lambda i: (i, 0)),
    )(pooled, Wp, b.reshape(1, NUM_CLASSES))


def kernel(utteranceTokens, embedding_table, W, b):
    tok = utteranceTokens.astype(jnp.int32)
    table_bf = embedding_table.astype(jnp.bfloat16)
    pooled = _pool(tok, table_bf)
    return _head(pooled, W, b)


# speedup vs baseline: 1.0876x; 1.0876x over previous
import functools

import jax
import jax.numpy as jnp
from jax import lax
from jax.experimental import pallas as pl
from jax.experimental.pallas import tpu as pltpu
from jax.experimental.pallas import tpu_sc as plsc

VOCAB = 1_000_000
EMBED_DIM = 64
NUM_CLASSES = 100
BATCH = 4096
SEQ_LEN = 200

NUM_CORES = 2
NUM_SUBCORES = 16
NUM_WORKERS = NUM_CORES * NUM_SUBCORES
B_PER_W = BATCH // NUM_WORKERS
LANES = 16

_SLICES = [(0, 104), (104, 96)]

_mesh = plsc.VectorSubcoreMesh(core_axis_name="c", subcore_axis_name="s")


@functools.partial(
    pl.kernel,
    mesh=_mesh,
    compiler_params=pltpu.CompilerParams(use_tc_tiling_on_sc=False),
    out_type=jax.ShapeDtypeStruct((BATCH, EMBED_DIM), jnp.float32),
    scratch_types=[
        pltpu.VMEM((B_PER_W, SEQ_LEN), jnp.int32),
        pltpu.VMEM((2, SEQ_LEN, EMBED_DIM), jnp.float32),
        pltpu.VMEM((B_PER_W, EMBED_DIM), jnp.float32),
        pltpu.SemaphoreType.DMA,
        pltpu.SemaphoreType.DMA,
    ],
)
def _pool(tok_hbm, table_hbm, out_hbm, idx_v, rows_v, out_v, sem0, sem1):
    wid = lax.axis_index("s") * NUM_CORES + lax.axis_index("c")
    base = wid * B_PER_W
    sems = (sem0, sem1)

    pltpu.sync_copy(tok_hbm.at[pl.ds(base, B_PER_W)], idx_v)

    def issue(u, buf):
        for off, n in _SLICES:
            pltpu.async_copy(
                table_hbm.at[idx_v.at[u, pl.ds(off, n)]],
                rows_v.at[buf, pl.ds(off, n)],
                sems[buf],
            )

    def drain(u, buf):
        for off, n in _SLICES:
            pltpu.make_async_copy(
                table_hbm.at[idx_v.at[u, pl.ds(off, n)]],
                rows_v.at[buf, pl.ds(off, n)],
                sems[buf],
            ).wait()

    zero = jnp.zeros((LANES,), jnp.float32)

    issue(0, 0)
    issue(1, 1)

    def outer(u0):
        for buf in range(2):
            u = u0 + buf
            drain(u, buf)
            rows_b = rows_v.at[buf]

            def acc_body(r, carry):
                return tuple(
                    carry[k] + rows_b[r, pl.ds(k * LANES, LANES)]
                    for k in range(4)
                )

            acc = lax.fori_loop(0, SEQ_LEN, acc_body, (zero,) * 4)
            for k in range(4):
                out_v[u, pl.ds(k * LANES, LANES)] = acc[k]

            @pl.when(u + 2 < B_PER_W)
            def _():
                issue(u + 2, buf)

    lax.fori_loop(0, B_PER_W // 2, lambda i, _: (outer(2 * i), 0)[1], 0)

    pltpu.sync_copy(out_v, out_hbm.at[pl.ds(base, B_PER_W)])


_RETILE_BLK = 8000


def _retile_body(i_ref, o_ref):
    x = i_ref[...] * jnp.float32(1.0 / SEQ_LEN)
    xr = x.reshape(_RETILE_BLK // 2, 2, EMBED_DIM)
    o_ref[...] = jnp.concatenate([xr[:, 0, :], xr[:, 1, :]], axis=1)


def _retile(table):
    return pl.pallas_call(
        _retile_body,
        out_shape=jax.ShapeDtypeStruct((VOCAB // 2, 2 * EMBED_DIM), jnp.float32),
        grid=(VOCAB // _RETILE_BLK,),
        in_specs=[pl.BlockSpec((_RETILE_BLK, EMBED_DIM), lambda i: (i, 0))],
        out_specs=pl.BlockSpec((_RETILE_BLK // 2, 2 * EMBED_DIM), lambda i: (i, 0)),
    )(table)


def _head_body(p_ref, w_ref, b_ref, o_ref):
    o_ref[...] = (
        jnp.dot(p_ref[...], w_ref[...], preferred_element_type=jnp.float32)
        + b_ref[...]
    )


def _head(pooled, W, b):
    blk = 512
    return pl.pallas_call(
        _head_body,
        out_shape=jax.ShapeDtypeStruct((BATCH, NUM_CLASSES), jnp.float32),
        grid=(BATCH // blk,),
        in_specs=[
            pl.BlockSpec((blk, EMBED_DIM), lambda i: (i, 0)),
            pl.BlockSpec((EMBED_DIM, NUM_CLASSES), lambda i: (0, 0)),
            pl.BlockSpec((1, NUM_CLASSES), lambda i: (0, 0)),
        ],
        out_specs=pl.BlockSpec((blk, NUM_CLASSES), lambda i: (i, 0)),
    )(pooled, W, b.reshape(1, NUM_CLASSES))


def kernel(utteranceTokens, embedding_table, W, b):
    tok = utteranceTokens.astype(jnp.int32)
    tbl_lin = _retile(embedding_table).reshape(VOCAB, EMBED_DIM)
    pooled = _pool(tok, tbl_lin)
    return _head(pooled, W, b)


# speedup vs baseline: 1.2674x; 1.1653x over previous
import functools

import jax
import jax.numpy as jnp
from jax import lax
from jax.experimental import pallas as pl
from jax.experimental.pallas import tpu as pltpu
from jax.experimental.pallas import tpu_sc as plsc

VOCAB = 1_000_000
EMBED_DIM = 64
NUM_CLASSES = 100
BATCH = 4096
SEQ_LEN = 200

NUM_CORES = 2
NUM_SUBCORES = 16
NUM_WORKERS = NUM_CORES * NUM_SUBCORES
B_PER_W = BATCH // NUM_WORKERS
LANES = 16

_SLICES = [(0, 104), (104, 96)]

_mesh = plsc.VectorSubcoreMesh(core_axis_name="c", subcore_axis_name="s")


@functools.partial(
    pl.kernel,
    mesh=_mesh,
    compiler_params=pltpu.CompilerParams(use_tc_tiling_on_sc=False),
    out_type=jax.ShapeDtypeStruct((BATCH, EMBED_DIM), jnp.float32),
    scratch_types=[
        pltpu.VMEM((B_PER_W, SEQ_LEN), jnp.int32),
        pltpu.VMEM((2, SEQ_LEN, EMBED_DIM), jnp.float32),
        pltpu.VMEM((B_PER_W, EMBED_DIM), jnp.float32),
        pltpu.SemaphoreType.DMA,
        pltpu.SemaphoreType.DMA,
    ],
)
def _pool(tok_hbm, table_hbm, out_hbm, idx_v, rows_v, out_v, sem0, sem1):
    wid = lax.axis_index("s") * NUM_CORES + lax.axis_index("c")
    base = wid * B_PER_W
    sems = (sem0, sem1)

    pltpu.sync_copy(tok_hbm.at[pl.ds(base, B_PER_W)], idx_v)

    def issue(u, buf):
        for off, n in _SLICES:
            pltpu.async_copy(
                table_hbm.at[idx_v.at[u, pl.ds(off, n)]],
                rows_v.at[buf, pl.ds(off, n)],
                sems[buf],
            )

    def drain(u, buf):
        for off, n in _SLICES:
            pltpu.make_async_copy(
                table_hbm.at[idx_v.at[u, pl.ds(off, n)]],
                rows_v.at[buf, pl.ds(off, n)],
                sems[buf],
            ).wait()

    zero = jnp.zeros((LANES,), jnp.float32)

    issue(0, 0)
    issue(1, 1)

    def outer(u0):
        for buf in range(2):
            u = u0 + buf
            drain(u, buf)
            rows_b = rows_v.at[buf]

            def acc_body(r, carry):
                return tuple(
                    carry[k] + rows_b[r, pl.ds(k * LANES, LANES)]
                    for k in range(4)
                )

            acc = lax.fori_loop(0, SEQ_LEN, acc_body, (zero,) * 4)
            for k in range(4):
                out_v[u, pl.ds(k * LANES, LANES)] = acc[k]

            @pl.when(u + 2 < B_PER_W)
            def _():
                issue(u + 2, buf)

    lax.fori_loop(0, B_PER_W // 2, lambda i, _: (outer(2 * i), 0)[1], 0)

    pltpu.sync_copy(out_v, out_hbm.at[pl.ds(base, B_PER_W)])


_RETILE_BLK = 4000


def _retile_body(a_ref, b_ref, o_ref):
    inv = jnp.float32(1.0 / SEQ_LEN)
    o_ref[...] = jnp.concatenate([a_ref[...] * inv, b_ref[...] * inv], axis=1)


def _retile(table):
    half_blocks = (VOCAB // 2) // _RETILE_BLK
    return pl.pallas_call(
        _retile_body,
        out_shape=jax.ShapeDtypeStruct((VOCAB // 2, 2 * EMBED_DIM), jnp.float32),
        grid=(half_blocks,),
        in_specs=[
            pl.BlockSpec((_RETILE_BLK, EMBED_DIM), lambda i: (i, 0)),
            pl.BlockSpec((_RETILE_BLK, EMBED_DIM),
                         lambda i: (i + half_blocks, 0)),
        ],
        out_specs=pl.BlockSpec((_RETILE_BLK, 2 * EMBED_DIM), lambda i: (i, 0)),
    )(table, table)


def _head_body(p_ref, w_ref, b_ref, o_ref):
    o_ref[...] = (
        jnp.dot(p_ref[...], w_ref[...], preferred_element_type=jnp.float32)
        + b_ref[...]
    )


def _head(pooled, W, b):
    blk = 512
    return pl.pallas_call(
        _head_body,
        out_shape=jax.ShapeDtypeStruct((BATCH, NUM_CLASSES), jnp.float32),
        grid=(BATCH // blk,),
        in_specs=[
            pl.BlockSpec((blk, EMBED_DIM), lambda i: (i, 0)),
            pl.BlockSpec((EMBED_DIM, NUM_CLASSES), lambda i: (0, 0)),
            pl.BlockSpec((1, NUM_CLASSES), lambda i: (0, 0)),
        ],
        out_specs=pl.BlockSpec((blk, NUM_CLASSES), lambda i: (i, 0)),
    )(pooled, W, b.reshape(1, NUM_CLASSES))


def kernel(utteranceTokens, embedding_table, W, b):
    tok = utteranceTokens.astype(jnp.int32)
    tok2 = jnp.where(tok < VOCAB // 2, 2 * tok, 2 * tok - (VOCAB - 1))
    tbl = lax.optimization_barrier(embedding_table)
    tbl_lin = _retile(tbl).reshape(VOCAB, EMBED_DIM)
    pooled = _pool(tok2, tbl_lin)
    return _head(pooled, W, b)


# speedup vs baseline: 2.2062x; 1.7407x over previous
import functools

import jax
import jax.numpy as jnp
from jax import lax
from jax.experimental import pallas as pl
from jax.experimental.pallas import tpu as pltpu
from jax.experimental.pallas import tpu_sc as plsc

VOCAB = 1_000_000
EMBED_DIM = 64
NUM_CLASSES = 100
BATCH = 4096
SEQ_LEN = 200

NUM_CORES = 2
NUM_SUBCORES = 16
NUM_WORKERS = NUM_CORES * NUM_SUBCORES
B_PER_W = BATCH // NUM_WORKERS
LANES = 16

_SLICES = [(0, 104), (104, 96)]

_mesh = plsc.VectorSubcoreMesh(core_axis_name="c", subcore_axis_name="s")


@functools.partial(
    pl.kernel,
    mesh=_mesh,
    compiler_params=pltpu.CompilerParams(use_tc_tiling_on_sc=False),
    out_type=jax.ShapeDtypeStruct((BATCH, EMBED_DIM), jnp.float32),
    scratch_types=[
        pltpu.VMEM((B_PER_W, SEQ_LEN), jnp.int32),
        pltpu.VMEM((2, SEQ_LEN, EMBED_DIM), jnp.float32),
        pltpu.VMEM((B_PER_W, EMBED_DIM), jnp.float32),
        pltpu.SemaphoreType.DMA,
        pltpu.SemaphoreType.DMA,
    ],
)
def _pool(tok_hbm, table_hbm, out_hbm, idx_v, rows_v, out_v, sem0, sem1):
    wid = lax.axis_index("s") * NUM_CORES + lax.axis_index("c")
    base = wid * B_PER_W
    sems = (sem0, sem1)

    pltpu.sync_copy(tok_hbm.at[pl.ds(base, B_PER_W)], idx_v)

    def issue(u, buf):
        for off, n in _SLICES:
            pltpu.async_copy(
                table_hbm.at[idx_v.at[u, pl.ds(off, n)]],
                rows_v.at[buf, pl.ds(off, n)],
                sems[buf],
            )

    def drain(u, buf):
        for off, n in _SLICES:
            pltpu.make_async_copy(
                table_hbm.at[idx_v.at[u, pl.ds(off, n)]],
                rows_v.at[buf, pl.ds(off, n)],
                sems[buf],
            ).wait()

    zero = jnp.zeros((LANES,), jnp.float32)

    issue(0, 0)
    issue(1, 1)

    def outer(u0):
        for buf in range(2):
            u = u0 + buf
            drain(u, buf)
            rows_b = rows_v.at[buf]

            def acc_body(r, carry):
                return tuple(
                    carry[k] + rows_b[r, pl.ds(k * LANES, LANES)]
                    for k in range(4)
                )

            acc = lax.fori_loop(0, SEQ_LEN, acc_body, (zero,) * 4)
            for k in range(4):
                out_v[u, pl.ds(k * LANES, LANES)] = acc[k]

            @pl.when(u + 2 < B_PER_W)
            def _():
                issue(u + 2, buf)

    lax.fori_loop(0, B_PER_W // 2, lambda i, _: (outer(2 * i), 0)[1], 0)

    pltpu.sync_copy(out_v, out_hbm.at[pl.ds(base, B_PER_W)])


_RETILE_BLK = 3968
_MAIN_STEPS = 126
_PAIR_OFF = _MAIN_STEPS * _RETILE_BLK
_TAIL_BLOCK = 252


def _retile_body(a_ref, b_ref, o_ref):
    i = pl.program_id(0)
    inv = jnp.float32(1.0 / SEQ_LEN)
    at = a_ref[...].T * inv

    @pl.when(i < _MAIN_STEPS)
    def _():
        bt = b_ref[...].T * inv
        o_ref[...] = jnp.concatenate([at, bt], axis=1)

    @pl.when(i == _MAIN_STEPS)
    def _():
        o_ref[...] = jnp.concatenate([at, jnp.roll(at, -32, axis=0)], axis=1)


def _retile(table_t):
    return pl.pallas_call(
        _retile_body,
        out_shape=jax.ShapeDtypeStruct((500032, 2 * EMBED_DIM), jnp.float32),
        grid=(_MAIN_STEPS + 1,),
        in_specs=[
            pl.BlockSpec(
                (EMBED_DIM, _RETILE_BLK),
                lambda i: (0, jnp.where(i < _MAIN_STEPS, i, _TAIL_BLOCK)),
            ),
            pl.BlockSpec(
                (EMBED_DIM, _RETILE_BLK),
                lambda i: (0, jnp.where(i < _MAIN_STEPS, i + _MAIN_STEPS,
                                        _TAIL_BLOCK)),
            ),
        ],
        out_specs=pl.BlockSpec((_RETILE_BLK, 2 * EMBED_DIM), lambda i: (i, 0)),
    )(table_t, table_t)


def _head_body(p_ref, w_ref, b_ref, o_ref):
    o_ref[...] = (
        jnp.dot(p_ref[...], w_ref[...], preferred_element_type=jnp.float32)
        + b_ref[...]
    )


def _head(pooled, W, b):
    blk = 512
    return pl.pallas_call(
        _head_body,
        out_shape=jax.ShapeDtypeStruct((BATCH, NUM_CLASSES), jnp.float32),
        grid=(BATCH // blk,),
        in_specs=[
            pl.BlockSpec((blk, EMBED_DIM), lambda i: (i, 0)),
            pl.BlockSpec((EMBED_DIM, NUM_CLASSES), lambda i: (0, 0)),
            pl.BlockSpec((1, NUM_CLASSES), lambda i: (0, 0)),
        ],
        out_specs=pl.BlockSpec((blk, NUM_CLASSES), lambda i: (i, 0)),
    )(pooled, W, b.reshape(1, NUM_CLASSES))


def kernel(utteranceTokens, embedding_table, W, b):
    t = utteranceTokens.astype(jnp.int32)
    d = t - 999936
    tok2 = jnp.where(
        t < 499968,
        2 * t,
        jnp.where(
            t < 999936,
            2 * t - 999935,
            jnp.where(d < 32, 999936 + 2 * d, 999873 + 2 * d),
        ),
    )
    tbl_lin = _retile(embedding_table.T).reshape(2 * 500032, EMBED_DIM)
    pooled = _pool(tok2, tbl_lin)
    return _head(pooled, W, b)


# speedup vs baseline: 2.5953x; 1.1764x over previous
import functools

import jax
import jax.numpy as jnp
from jax import lax
from jax.experimental import pallas as pl
from jax.experimental.pallas import tpu as pltpu
from jax.experimental.pallas import tpu_sc as plsc

VOCAB = 1_000_000
EMBED_DIM = 64
NUM_CLASSES = 100
BATCH = 4096
SEQ_LEN = 200

NUM_CORES = 2
NUM_SUBCORES = 16
NUM_WORKERS = NUM_CORES * NUM_SUBCORES
B_PER_W = BATCH // NUM_WORKERS
LANES = 16

_SLICES = [(0, 104), (104, 96)]

_mesh = plsc.VectorSubcoreMesh(core_axis_name="c", subcore_axis_name="s")


@functools.partial(
    pl.kernel,
    mesh=_mesh,
    compiler_params=pltpu.CompilerParams(use_tc_tiling_on_sc=False),
    out_type=jax.ShapeDtypeStruct((BATCH, EMBED_DIM), jnp.float32),
    scratch_types=[
        pltpu.VMEM((B_PER_W, SEQ_LEN), jnp.int32),
        pltpu.VMEM((2, SEQ_LEN, EMBED_DIM), jnp.float32),
        pltpu.VMEM((B_PER_W, EMBED_DIM), jnp.float32),
        pltpu.SemaphoreType.DMA,
        pltpu.SemaphoreType.DMA,
    ],
)
def _pool(tok_hbm, table_hbm, out_hbm, idx_v, rows_v, out_v, sem0, sem1):
    wid = lax.axis_index("s") * NUM_CORES + lax.axis_index("c")
    base = wid * B_PER_W
    sems = (sem0, sem1)

    pltpu.sync_copy(tok_hbm.at[pl.ds(base, B_PER_W)], idx_v)

    def issue(u, buf):
        for off, n in _SLICES:
            pltpu.async_copy(
                table_hbm.at[idx_v.at[u, pl.ds(off, n)]],
                rows_v.at[buf, pl.ds(off, n)],
                sems[buf],
            )

    def drain(u, buf):
        for off, n in _SLICES:
            pltpu.make_async_copy(
                table_hbm.at[idx_v.at[u, pl.ds(off, n)]],
                rows_v.at[buf, pl.ds(off, n)],
                sems[buf],
            ).wait()

    zero = jnp.zeros((LANES,), jnp.float32)

    issue(0, 0)
    issue(1, 1)

    def outer(u0):
        for buf in range(2):
            u = u0 + buf
            drain(u, buf)
            rows_b = rows_v.at[buf]

            def acc_body(r, carry):
                return tuple(
                    carry[k] + rows_b[r, pl.ds(k * LANES, LANES)]
                    for k in range(4)
                )

            acc = lax.fori_loop(0, SEQ_LEN, acc_body, (zero,) * 4)
            for k in range(4):
                out_v[u, pl.ds(k * LANES, LANES)] = acc[k]

            @pl.when(u + 2 < B_PER_W)
            def _():
                issue(u + 2, buf)

    lax.fori_loop(0, B_PER_W // 2, lambda i, _: (outer(2 * i), 0)[1], 0)

    pltpu.sync_copy(out_v, out_hbm.at[pl.ds(base, B_PER_W)])


_RETILE_BLK = 3968
_MAIN_STEPS = 126
_PAIR_OFF = _MAIN_STEPS * _RETILE_BLK
_TAIL_BLOCK = 252


def _retile_body(a_ref, b_ref, o_ref):
    i = pl.program_id(0)
    inv = jnp.float32(1.0 / SEQ_LEN)
    a = a_ref[...]

    @pl.when(i < _MAIN_STEPS)
    def _():
        z = jnp.concatenate([a, b_ref[...]], axis=0)
        o_ref[...] = z.T * inv

    @pl.when(i == _MAIN_STEPS)
    def _():
        z = jnp.concatenate([a, jnp.roll(a, -32, axis=1)], axis=0)
        o_ref[...] = z.T * inv


def _retile(table_t):
    return pl.pallas_call(
        _retile_body,
        out_shape=jax.ShapeDtypeStruct((500032, 2 * EMBED_DIM), jnp.float32),
        grid=(_MAIN_STEPS + 1,),
        in_specs=[
            pl.BlockSpec(
                (EMBED_DIM, _RETILE_BLK),
                lambda i: (0, jnp.where(i < _MAIN_STEPS, i, _TAIL_BLOCK)),
            ),
            pl.BlockSpec(
                (EMBED_DIM, _RETILE_BLK),
                lambda i: (0, jnp.where(i < _MAIN_STEPS, i + _MAIN_STEPS,
                                        _TAIL_BLOCK)),
            ),
        ],
        out_specs=pl.BlockSpec((_RETILE_BLK, 2 * EMBED_DIM), lambda i: (i, 0)),
    )(table_t, table_t)


def _head_body(p_ref, w_ref, b_ref, o_ref):
    o_ref[...] = (
        jnp.dot(p_ref[...], w_ref[...], preferred_element_type=jnp.float32)
        + b_ref[...]
    )


def _head(pooled, W, b):
    blk = 512
    return pl.pallas_call(
        _head_body,
        out_shape=jax.ShapeDtypeStruct((BATCH, NUM_CLASSES), jnp.float32),
        grid=(BATCH // blk,),
        in_specs=[
            pl.BlockSpec((blk, EMBED_DIM), lambda i: (i, 0)),
            pl.BlockSpec((EMBED_DIM, NUM_CLASSES), lambda i: (0, 0)),
            pl.BlockSpec((1, NUM_CLASSES), lambda i: (0, 0)),
        ],
        out_specs=pl.BlockSpec((blk, NUM_CLASSES), lambda i: (i, 0)),
    )(pooled, W, b.reshape(1, NUM_CLASSES))


def kernel(utteranceTokens, embedding_table, W, b):
    t = utteranceTokens.astype(jnp.int32)
    d = t - 999936
    tok2 = jnp.where(
        t < 499968,
        2 * t,
        jnp.where(
            t < 999936,
            2 * t - 999935,
            jnp.where(d < 32, 999936 + 2 * d, 999873 + 2 * d),
        ),
    )
    tbl_lin = _retile(embedding_table.T).reshape(2 * 500032, EMBED_DIM)
    pooled = _pool(tok2, tbl_lin)
    return _head(pooled, W, b)


# speedup vs baseline: 2.8408x; 1.0946x over previous
import functools

import jax
import jax.numpy as jnp
from jax import lax
from jax.experimental import pallas as pl
from jax.experimental.pallas import tpu as pltpu
from jax.experimental.pallas import tpu_sc as plsc

VOCAB = 1_000_000
EMBED_DIM = 64
NUM_CLASSES = 100
BATCH = 4096
SEQ_LEN = 200

NUM_CORES = 2
NUM_SUBCORES = 16
NUM_WORKERS = NUM_CORES * NUM_SUBCORES
B_PER_W = BATCH // NUM_WORKERS
LANES = 16

_SLICES = [(0, 104), (104, 96)]

_mesh = plsc.VectorSubcoreMesh(core_axis_name="c", subcore_axis_name="s")


@functools.partial(
    pl.kernel,
    mesh=_mesh,
    compiler_params=pltpu.CompilerParams(use_tc_tiling_on_sc=False),
    out_type=jax.ShapeDtypeStruct((BATCH, EMBED_DIM), jnp.float32),
    scratch_types=[
        pltpu.VMEM((B_PER_W, SEQ_LEN), jnp.int32),
        pltpu.VMEM((2, SEQ_LEN, EMBED_DIM), jnp.float32),
        pltpu.VMEM((B_PER_W, EMBED_DIM), jnp.float32),
        pltpu.SemaphoreType.DMA,
        pltpu.SemaphoreType.DMA,
    ],
)
def _pool(tok_hbm, table_hbm, out_hbm, idx_v, rows_v, out_v, sem0, sem1):
    wid = lax.axis_index("s") * NUM_CORES + lax.axis_index("c")
    base = wid * B_PER_W
    sems = (sem0, sem1)

    pltpu.sync_copy(tok_hbm.at[pl.ds(base, B_PER_W)], idx_v)

    def issue(u, buf):
        for off, n in _SLICES:
            pltpu.async_copy(
                table_hbm.at[idx_v.at[u, pl.ds(off, n)]],
                rows_v.at[buf, pl.ds(off, n)],
                sems[buf],
            )

    def drain(u, buf):
        for off, n in _SLICES:
            pltpu.make_async_copy(
                table_hbm.at[idx_v.at[u, pl.ds(off, n)]],
                rows_v.at[buf, pl.ds(off, n)],
                sems[buf],
            ).wait()

    zero = jnp.zeros((LANES,), jnp.float32)

    issue(0, 0)
    issue(1, 1)

    def outer(u0):
        for buf in range(2):
            u = u0 + buf
            drain(u, buf)
            rows_b = rows_v.at[buf]

            def acc_body(r, carry):
                return tuple(
                    carry[k] + rows_b[r, pl.ds(k * LANES, LANES)]
                    for k in range(4)
                )

            acc = lax.fori_loop(0, SEQ_LEN, acc_body, (zero,) * 4)
            for k in range(4):
                out_v[u, pl.ds(k * LANES, LANES)] = acc[k]

            @pl.when(u + 2 < B_PER_W)
            def _():
                issue(u + 2, buf)

    lax.fori_loop(0, B_PER_W // 2, lambda i, _: (outer(2 * i), 0)[1], 0)

    pltpu.sync_copy(out_v, out_hbm.at[pl.ds(base, B_PER_W)])


_RETILE_BLK = 7936
_MAIN_STEPS = 63
_PAIR_OFF = _MAIN_STEPS * _RETILE_BLK
_TAIL_BLOCK = 126


def _retile_body(a_ref, b_ref, o_ref):
    i = pl.program_id(0)
    inv = jnp.float32(1.0 / SEQ_LEN)
    a = a_ref[...]

    @pl.when(i < _MAIN_STEPS)
    def _():
        z = jnp.concatenate([a, b_ref[...]], axis=0)
        o_ref[...] = z.T * inv

    @pl.when(i == _MAIN_STEPS)
    def _():
        z = jnp.concatenate([a, jnp.roll(a, -32, axis=1)], axis=0)
        o_ref[...] = z.T * inv


def _retile(table_t):
    return pl.pallas_call(
        _retile_body,
        out_shape=jax.ShapeDtypeStruct((500032, 2 * EMBED_DIM), jnp.float32),
        grid=(_MAIN_STEPS + 1,),
        in_specs=[
            pl.BlockSpec(
                (EMBED_DIM, _RETILE_BLK),
                lambda i: (0, jnp.where(i < _MAIN_STEPS, i, _TAIL_BLOCK)),
            ),
            pl.BlockSpec(
                (EMBED_DIM, _RETILE_BLK),
                lambda i: (0, jnp.where(i < _MAIN_STEPS, i + _MAIN_STEPS,
                                        _TAIL_BLOCK)),
            ),
        ],
        out_specs=pl.BlockSpec((_RETILE_BLK, 2 * EMBED_DIM), lambda i: (i, 0)),
    )(table_t, table_t)


def _head_body(p_ref, w_ref, b_ref, o_ref):
    o_ref[...] = (
        jnp.dot(p_ref[...], w_ref[...], preferred_element_type=jnp.float32)
        + b_ref[...]
    )


def _head(pooled, W, b):
    blk = 512
    return pl.pallas_call(
        _head_body,
        out_shape=jax.ShapeDtypeStruct((BATCH, NUM_CLASSES), jnp.float32),
        grid=(BATCH // blk,),
        in_specs=[
            pl.BlockSpec((blk, EMBED_DIM), lambda i: (i, 0)),
            pl.BlockSpec((EMBED_DIM, NUM_CLASSES), lambda i: (0, 0)),
            pl.BlockSpec((1, NUM_CLASSES), lambda i: (0, 0)),
        ],
        out_specs=pl.BlockSpec((blk, NUM_CLASSES), lambda i: (i, 0)),
    )(pooled, W, b.reshape(1, NUM_CLASSES))


def kernel(utteranceTokens, embedding_table, W, b):
    t = utteranceTokens.astype(jnp.int32)
    d = t - 999936
    tok2 = jnp.where(
        t < 499968,
        2 * t,
        jnp.where(
            t < 999936,
            2 * t - 999935,
            jnp.where(d < 32, 999936 + 2 * d, 999873 + 2 * d),
        ),
    )
    tbl_lin = _retile(embedding_table.T).reshape(2 * 500032, EMBED_DIM)
    pooled = _pool(tok2, tbl_lin)
    return _head(pooled, W, b)


# speedup vs baseline: 2.8891x; 1.0170x over previous
import functools

import jax
import jax.numpy as jnp
from jax import lax
from jax.experimental import pallas as pl
from jax.experimental.pallas import tpu as pltpu
from jax.experimental.pallas import tpu_sc as plsc

VOCAB = 1_000_000
EMBED_DIM = 64
NUM_CLASSES = 100
BATCH = 4096
SEQ_LEN = 200

NUM_CORES = 2
NUM_SUBCORES = 16
NUM_WORKERS = NUM_CORES * NUM_SUBCORES
B_PER_W = BATCH // NUM_WORKERS
LANES = 16

_SLICES = [(0, 104), (104, 96)]

_mesh = plsc.VectorSubcoreMesh(core_axis_name="c", subcore_axis_name="s")


@functools.partial(
    pl.kernel,
    mesh=_mesh,
    compiler_params=pltpu.CompilerParams(use_tc_tiling_on_sc=False),
    out_type=jax.ShapeDtypeStruct((BATCH, EMBED_DIM), jnp.float32),
    scratch_types=[
        pltpu.VMEM((B_PER_W, SEQ_LEN), jnp.int32),
        pltpu.VMEM((2, SEQ_LEN, EMBED_DIM), jnp.float32),
        pltpu.VMEM((B_PER_W, EMBED_DIM), jnp.float32),
        pltpu.SemaphoreType.DMA,
        pltpu.SemaphoreType.DMA,
    ],
)
def _pool(tok_hbm, table_hbm, out_hbm, idx_v, rows_v, out_v, sem0, sem1):
    wid = lax.axis_index("s") * NUM_CORES + lax.axis_index("c")
    base = wid * B_PER_W
    sems = (sem0, sem1)

    pltpu.sync_copy(tok_hbm.at[pl.ds(base, B_PER_W)], idx_v)

    def issue(u, buf):
        for off, n in _SLICES:
            pltpu.async_copy(
                table_hbm.at[idx_v.at[u, pl.ds(off, n)]],
                rows_v.at[buf, pl.ds(off, n)],
                sems[buf],
            )

    def drain(u, buf):
        for off, n in _SLICES:
            pltpu.make_async_copy(
                table_hbm.at[idx_v.at[u, pl.ds(off, n)]],
                rows_v.at[buf, pl.ds(off, n)],
                sems[buf],
            ).wait()

    zero = jnp.zeros((LANES,), jnp.float32)

    issue(0, 0)
    issue(1, 1)

    def outer(u0):
        for buf in range(2):
            u = u0 + buf
            drain(u, buf)
            rows_b = rows_v.at[buf]

            def acc_body(r, carry):
                return tuple(
                    carry[k] + rows_b[r, pl.ds(k * LANES, LANES)]
                    for k in range(4)
                )

            acc = lax.fori_loop(0, SEQ_LEN, acc_body, (zero,) * 4)
            for k in range(4):
                out_v[u, pl.ds(k * LANES, LANES)] = acc[k]

            @pl.when(u + 2 < B_PER_W)
            def _():
                issue(u + 2, buf)

    lax.fori_loop(0, B_PER_W // 2, lambda i, _: (outer(2 * i), 0)[1], 0)

    pltpu.sync_copy(out_v, out_hbm.at[pl.ds(base, B_PER_W)])


_RETILE_BLK = 11904
_MAIN_STEPS = 42
_PAIR_OFF = _MAIN_STEPS * _RETILE_BLK
_TAIL_BLOCK = 84


def _retile_body(a_ref, b_ref, o_ref):
    i = pl.program_id(0)
    inv = jnp.float32(1.0 / SEQ_LEN)
    a = a_ref[...]

    @pl.when(i < _MAIN_STEPS)
    def _():
        z = jnp.concatenate([a, b_ref[...]], axis=0)
        o_ref[...] = z.T * inv

    @pl.when(i == _MAIN_STEPS)
    def _():
        z = jnp.concatenate([a, jnp.roll(a, -32, axis=1)], axis=0)
        o_ref[...] = z.T * inv


def _retile(table_t):
    return pl.pallas_call(
        _retile_body,
        out_shape=jax.ShapeDtypeStruct((500032, 2 * EMBED_DIM), jnp.float32),
        grid=(_MAIN_STEPS + 1,),
        in_specs=[
            pl.BlockSpec(
                (EMBED_DIM, _RETILE_BLK),
                lambda i: (0, jnp.where(i < _MAIN_STEPS, i, _TAIL_BLOCK)),
            ),
            pl.BlockSpec(
                (EMBED_DIM, _RETILE_BLK),
                lambda i: (0, jnp.where(i < _MAIN_STEPS, i + _MAIN_STEPS,
                                        _TAIL_BLOCK)),
            ),
        ],
        out_specs=pl.BlockSpec((_RETILE_BLK, 2 * EMBED_DIM), lambda i: (i, 0)),
    )(table_t, table_t)


def _head_body(p_ref, w_ref, b_ref, o_ref):
    o_ref[...] = (
        jnp.dot(p_ref[...], w_ref[...], preferred_element_type=jnp.float32)
        + b_ref[...]
    )


def _head(pooled, W, b):
    blk = 512
    return pl.pallas_call(
        _head_body,
        out_shape=jax.ShapeDtypeStruct((BATCH, NUM_CLASSES), jnp.float32),
        grid=(BATCH // blk,),
        in_specs=[
            pl.BlockSpec((blk, EMBED_DIM), lambda i: (i, 0)),
            pl.BlockSpec((EMBED_DIM, NUM_CLASSES), lambda i: (0, 0)),
            pl.BlockSpec((1, NUM_CLASSES), lambda i: (0, 0)),
        ],
        out_specs=pl.BlockSpec((blk, NUM_CLASSES), lambda i: (i, 0)),
    )(pooled, W, b.reshape(1, NUM_CLASSES))


def kernel(utteranceTokens, embedding_table, W, b):
    t = utteranceTokens.astype(jnp.int32)
    d = t - 999936
    tok2 = jnp.where(
        t < 499968,
        2 * t,
        jnp.where(
            t < 999936,
            2 * t - 999935,
            jnp.where(d < 32, 999936 + 2 * d, 999873 + 2 * d),
        ),
    )
    tbl_lin = _retile(embedding_table.T).reshape(2 * 500032, EMBED_DIM)
    pooled = _pool(tok2, tbl_lin)
    return _head(pooled, W, b)
